# baseline (speedup 1.0000x reference)
import jax
import jax.numpy as jnp
from jax import lax
from jax.experimental import pallas as pl
from jax.experimental.pallas import tpu as pltpu
from jax.experimental.pallas import tpu_sc as plsc

_B, _N, _D = 16, 196, 768
_H, _E, _K, _DM = 16, 64, 2048, 64
_EA = _E + 8
_BN = _B * _N
_BNP = 3328
_T = 256
_NC = 4
_KC = _K // _NC // 8
_PREC = jax.lax.Precision.DEFAULT

_NW = 32
_TPW = _BNP // _NW
_R0, _R1 = 56, 48


def _prep_block(cb_ref, val_ref, cba_ref, vp_ref):
    vp_ref[:, 0:_DM] = val_ref[0]
    vp_ref[:, _DM:128] = jnp.zeros((_K, 128 - _DM), jnp.float32)
    cb = cb_ref[0]
    esq = jnp.sum(cb * cb, axis=1)
    e1 = esq.astype(jnp.bfloat16).astype(jnp.float32)
    r1 = esq - e1
    e2 = r1.astype(jnp.bfloat16).astype(jnp.float32)
    r2 = r1 - e2
    e3 = r2.astype(jnp.bfloat16).astype(jnp.float32)
    e4 = (r2 - e3).astype(jnp.bfloat16).astype(jnp.float32)
    z = jnp.zeros((_K,), jnp.float32)
    cba_ref[0, :, 0:_E] = cb
    cba_ref[0, :, _E:_EA] = jnp.stack(
        [-e1, -e2, -e3, -e4, z, z, z, z], axis=1)


def _main_block(emb_ref, rp_ref, cba_ref, idx_ref, pcat_ref, x2t_ref,
                score0_ref, score1_ref):
    @pl.when(pl.program_id(0) == 0)
    def _():
        pcat_ref[...] = 2.0 * jnp.concatenate(
            [rp_ref[hh] for hh in range(_H)], axis=1)
        for hh in range(_H):
            x2t_ref[hh * _EA + _E:hh * _EA + _E + 4, :] = jnp.ones(
                (4, _T), jnp.float32)
            x2t_ref[hh * _EA + _E + 4:(hh + 1) * _EA, :] = jnp.zeros(
                (4, _T), jnp.float32)

    x2 = jnp.dot(emb_ref[...], pcat_ref[...], precision=_PREC)
    for hh in range(_H):
        x2t_ref[hh * _EA:hh * _EA + _E, :] = x2[:, hh * _E:(hh + 1) * _E].T

    sub_iota = lax.broadcasted_iota(jnp.int32, (8, _T), 0)
    pieces = []
    for h in range(_H):
        score_ref = score0_ref if h % 2 == 0 else score1_ref
        score_ref[...] = jnp.dot(
            cba_ref[h], x2t_ref[pl.ds(h * _EA, _EA), :], precision=_PREC)

        def body(kt, carry, score_ref=score_ref):
            out = []
            for g in range(_NC):
                best, bestkt = carry[2 * g], carry[2 * g + 1]
                tile = score_ref[pl.ds((g * _KC + kt) * 8, 8), :]
                upd = tile > best
                out.append(jnp.where(upd, tile, best))
                out.append(jnp.where(upd, kt, bestkt))
            return tuple(out)

        init = []
        for g in range(_NC):
            init.append(score_ref[pl.ds(g * _KC * 8, 8), :])
            init.append(jnp.zeros((8, _T), jnp.int32))
        res = tuple(init)
        for kt in range(1, _KC):
            res = body(kt, res)

        best = res[0]
        best_k = res[1] * 8 + sub_iota
        for g in range(1, _NC):
            v, kt_g = res[2 * g], res[2 * g + 1]
            kg = (kt_g + g * _KC) * 8 + sub_iota
            upd = v > best
            best = jnp.where(upd, v, best)
            best_k = jnp.where(upd, kg, best_k)
        mx = jnp.max(best, axis=0, keepdims=True)
        cand = jnp.where(best == mx, best_k, _K * _H)
        pieces.append(jnp.min(cand, axis=0) + h * _K)
    idx_ref[...] = jnp.stack(pieces, axis=0).T


_TPR = _TPW // 2


def _sc_gather_mean(vals_hbm, idx_hbm, out_hbm, idx_v, rows_v, out_v, sem):
    wid = lax.axis_index("s") * 2 + lax.axis_index("c")
    base = wid * _TPW
    pltpu.sync_copy(idx_hbm.at[pl.ds(base * _H, _TPW * _H)], idx_v)

    for r in range(2):
        pltpu.async_copy(
            vals_hbm.at[idx_v.at[pl.ds(r * _TPR * _H, _TPR * _H)]],
            rows_v, sem).wait()

        @pl.loop(0, _TPR)
        def _(t, r=r):
            for c in range(0, _DM, 16):
                acc = rows_v.at[t * _H, pl.ds(c, 16)][...]
                for h in range(1, _H):
                    acc = acc + rows_v.at[t * _H + h, pl.ds(c, 16)][...]
                out_v.at[pl.ds((r * _TPR + t) * _DM + c, 16)][...] = (
                    acc * (1.0 / _H))

    pltpu.sync_copy(out_v, out_hbm.at[pl.ds(base * _DM, _TPW * _DM)])


def kernel(embeddings, rand_proj, codebook, values):
    emb = jnp.pad(embeddings.reshape(_BN, _D), ((0, _BNP - _BN), (0, 0)))

    cba, vals_flat = pl.pallas_call(
        _prep_block,
        grid=(_H,),
        in_specs=[
            pl.BlockSpec((1, _K, _E), lambda h: (h, 0, 0)),
            pl.BlockSpec((1, _K, _DM), lambda h: (h, 0, 0)),
        ],
        out_specs=[
            pl.BlockSpec((1, _K, _EA), lambda h: (h, 0, 0)),
            pl.BlockSpec((_K, 128), lambda h: (h, 0)),
        ],
        out_shape=[
            jax.ShapeDtypeStruct((_H, _K, _EA), jnp.float32),
            jax.ShapeDtypeStruct((_H * _K, 128), jnp.float32),
        ],
        compiler_params=pltpu.CompilerParams(
            dimension_semantics=("arbitrary",)),
    )(codebook, values)

    idx3 = pl.pallas_call(
        _main_block,
        grid=(_BNP // _T,),
        in_specs=[
            pl.BlockSpec((_T, _D), lambda i: (i, 0)),
            pl.BlockSpec((_H, _D, _E), lambda i: (0, 0, 0)),
            pl.BlockSpec((_H, _K, _EA), lambda i: (0, 0, 0)),
        ],
        out_specs=pl.BlockSpec((_T, _H), lambda i: (i, 0)),
        out_shape=jax.ShapeDtypeStruct((_BNP, _H), jnp.int32),
        scratch_shapes=[
            pltpu.VMEM((_D, _H * _E), jnp.float32),
            pltpu.VMEM((_H * _EA, _T), jnp.float32),
            pltpu.VMEM((_K, _T), jnp.float32),
            pltpu.VMEM((_K, _T), jnp.float32),
        ],
        compiler_params=pltpu.CompilerParams(
            dimension_semantics=("arbitrary",)),
    )(emb, rand_proj, cba)

    idx_flat = idx3.reshape(_BNP * _H)
    mesh = plsc.VectorSubcoreMesh(core_axis_name="c", subcore_axis_name="s")
    sc = pl.kernel(
        _sc_gather_mean,
        mesh=mesh,
        out_type=jax.ShapeDtypeStruct((_BNP * _DM,), jnp.float32),
        scratch_types=[
            pltpu.VMEM((_TPW * _H,), jnp.int32),
            pltpu.VMEM((_TPR * _H, 128), jnp.float32),
            pltpu.VMEM((_TPW * _DM,), jnp.float32),
            pltpu.SemaphoreType.DMA,
        ],
    )
    out = sc(vals_flat, idx_flat)
    return out.reshape(_BNP, _DM)[:_BN].reshape(_B, _N, _DM)


# speedup vs baseline: 1.0297x; 1.0297x over previous
import jax
import jax.numpy as jnp
from jax import lax
from jax.experimental import pallas as pl
from jax.experimental.pallas import tpu as pltpu
from jax.experimental.pallas import tpu_sc as plsc

_B, _N, _D = 16, 196, 768
_H, _E, _K, _DM = 16, 64, 2048, 64
_EA = _E + 8
_BN = _B * _N
_BNP = 3328
_T = 256
_NC = 4
_KC = _K // _NC // 8
_PREC = jax.lax.Precision.DEFAULT

_NW = 32
_TPW = _BNP // _NW
_R0, _R1 = 56, 48


def _prep_block(cb_ref, cba_ref):
    cb = cb_ref[0]
    esq = jnp.sum(cb * cb, axis=1)
    e1 = esq.astype(jnp.bfloat16).astype(jnp.float32)
    r1 = esq - e1
    e2 = r1.astype(jnp.bfloat16).astype(jnp.float32)
    r2 = r1 - e2
    e3 = r2.astype(jnp.bfloat16).astype(jnp.float32)
    e4 = (r2 - e3).astype(jnp.bfloat16).astype(jnp.float32)
    z = jnp.zeros((_K,), jnp.float32)
    cba_ref[0, :, 0:_E] = cb
    cba_ref[0, :, _E:_EA] = jnp.stack(
        [-e1, -e2, -e3, -e4, z, z, z, z], axis=1)


def _main_block(emb_ref, rp_ref, cba_ref, idx_ref, pcat_ref, x2t_ref,
                score0_ref, score1_ref):
    @pl.when(pl.program_id(0) == 0)
    def _():
        pcat_ref[...] = 2.0 * jnp.concatenate(
            [rp_ref[hh] for hh in range(_H)], axis=1)
        for hh in range(_H):
            x2t_ref[hh * _EA + _E:hh * _EA + _E + 4, :] = jnp.ones(
                (4, _T), jnp.float32)
            x2t_ref[hh * _EA + _E + 4:(hh + 1) * _EA, :] = jnp.zeros(
                (4, _T), jnp.float32)

    x2 = jnp.dot(emb_ref[...], pcat_ref[...], precision=_PREC)
    for hh in range(_H):
        x2t_ref[hh * _EA:hh * _EA + _E, :] = x2[:, hh * _E:(hh + 1) * _E].T

    sub_iota = lax.broadcasted_iota(jnp.int32, (8, _T), 0)
    pieces = []
    for h in range(_H):
        score_ref = score0_ref if h % 2 == 0 else score1_ref
        score_ref[...] = jnp.dot(
            cba_ref[h], x2t_ref[pl.ds(h * _EA, _EA), :], precision=_PREC)

        def body(kt, carry, score_ref=score_ref):
            out = []
            for g in range(_NC):
                best, bestkt = carry[2 * g], carry[2 * g + 1]
                tile = score_ref[pl.ds((g * _KC + kt) * 8, 8), :]
                upd = tile > best
                out.append(jnp.where(upd, tile, best))
                out.append(jnp.where(upd, kt, bestkt))
            return tuple(out)

        init = []
        for g in range(_NC):
            init.append(score_ref[pl.ds(g * _KC * 8, 8), :])
            init.append(jnp.zeros((8, _T), jnp.int32))
        res = tuple(init)
        for kt in range(1, _KC):
            res = body(kt, res)

        best = res[0]
        best_k = res[1] * 8 + sub_iota
        for g in range(1, _NC):
            v, kt_g = res[2 * g], res[2 * g + 1]
            kg = (kt_g + g * _KC) * 8 + sub_iota
            upd = v > best
            best = jnp.where(upd, v, best)
            best_k = jnp.where(upd, kg, best_k)
        mx = jnp.max(best, axis=0, keepdims=True)
        cand = jnp.where(best == mx, best_k, _K * _H)
        pieces.append(jnp.min(cand, axis=0) + h * _K)
    idx_ref[...] = jnp.stack(pieces, axis=0).T


_TPR = _TPW // 2


def _sc_gather_mean(vals_hbm, idx_hbm, out_hbm, idx_v, rows_v, out_v, sem):
    wid = lax.axis_index("s") * 2 + lax.axis_index("c")
    base = wid * _TPW
    pltpu.sync_copy(idx_hbm.at[pl.ds(base * _H, _TPW * _H)], idx_v)

    for r in range(2):
        pltpu.async_copy(
            vals_hbm.at[idx_v.at[pl.ds(r * _TPR * _H, _TPR * _H)]],
            rows_v, sem).wait()

        @pl.loop(0, _TPR)
        def _(t, r=r):
            for c in range(0, _DM, 16):
                acc = rows_v.at[t * _H, pl.ds(c, 16)][...]
                for h in range(1, _H):
                    acc = acc + rows_v.at[t * _H + h, pl.ds(c, 16)][...]
                out_v.at[pl.ds((r * _TPR + t) * _DM + c, 16)][...] = (
                    acc * (1.0 / _H))

    pltpu.sync_copy(out_v, out_hbm.at[pl.ds(base * _DM, _TPW * _DM)])


def kernel(embeddings, rand_proj, codebook, values):
    emb = jnp.pad(embeddings.reshape(_BN, _D), ((0, _BNP - _BN), (0, 0)))

    cba = pl.pallas_call(
        _prep_block,
        grid=(_H,),
        in_specs=[pl.BlockSpec((1, _K, _E), lambda h: (h, 0, 0))],
        out_specs=pl.BlockSpec((1, _K, _EA), lambda h: (h, 0, 0)),
        out_shape=jax.ShapeDtypeStruct((_H, _K, _EA), jnp.float32),
        compiler_params=pltpu.CompilerParams(
            dimension_semantics=("arbitrary",)),
    )(codebook)

    idx3 = pl.pallas_call(
        _main_block,
        grid=(_BNP // _T,),
        in_specs=[
            pl.BlockSpec((_T, _D), lambda i: (i, 0)),
            pl.BlockSpec((_H, _D, _E), lambda i: (0, 0, 0)),
            pl.BlockSpec((_H, _K, _EA), lambda i: (0, 0, 0)),
        ],
        out_specs=pl.BlockSpec((_T, _H), lambda i: (i, 0)),
        out_shape=jax.ShapeDtypeStruct((_BNP, _H), jnp.int32),
        scratch_shapes=[
            pltpu.VMEM((_D, _H * _E), jnp.float32),
            pltpu.VMEM((_H * _EA, _T), jnp.float32),
            pltpu.VMEM((_K, _T), jnp.float32),
            pltpu.VMEM((_K, _T), jnp.float32),
        ],
        compiler_params=pltpu.CompilerParams(
            dimension_semantics=("arbitrary",)),
    )(emb, rand_proj, cba)

    vals_flat = jnp.pad(values.reshape(_H * _K, _DM), ((0, 0), (0, 128 - _DM)))
    idx_flat = idx3.reshape(_BNP * _H)
    mesh = plsc.VectorSubcoreMesh(core_axis_name="c", subcore_axis_name="s")
    sc = pl.kernel(
        _sc_gather_mean,
        mesh=mesh,
        out_type=jax.ShapeDtypeStruct((_BNP * _DM,), jnp.float32),
        scratch_types=[
            pltpu.VMEM((_TPW * _H,), jnp.int32),
            pltpu.VMEM((_TPR * _H, 128), jnp.float32),
            pltpu.VMEM((_TPW * _DM,), jnp.float32),
            pltpu.SemaphoreType.DMA,
        ],
    )
    out = sc(vals_flat, idx_flat)
    return out.reshape(_BNP, _DM)[:_BN].reshape(_B, _N, _DM)
